# Initial kernel scaffold; baseline (speedup 1.0000x reference)
import jax
import jax.numpy as jnp
from jax.experimental import pallas as pl


def kernel(ctr, table):
    raise NotImplementedError("write your pallas kernel here")


# speedup vs baseline: 7.5616x; 7.5616x over previous
import functools

import jax
import jax.numpy as jnp
from jax import lax
from jax.experimental import pallas as pl
from jax.experimental.pallas import tpu as pltpu
from jax.experimental.pallas import tpu_sc as plsc

LANES = 16
IDX_W = 128


def _sc_workers():
    try:
        info = plsc.get_sparse_core_info()
        return info.num_cores, info.num_subcores
    except Exception:
        return 2, 16


def kernel(ctr, table):
    batch, clicked = ctr.shape
    vocab, d = table.shape
    n_total = batch * clicked

    nc, ns = _sc_workers()
    nw = nc * ns
    per_w = n_total // nw
    assert per_w * nw == n_total and per_w % IDX_W == 0

    chunk = 256
    if per_w % chunk != 0:
        chunk = IDX_W
    k_streams = chunk // IDX_W
    n_chunks = per_w // chunk
    per_w_rows = per_w // IDX_W

    idx2d = ctr.reshape(n_total // IDX_W, IDX_W)

    mesh = plsc.VectorSubcoreMesh(core_axis_name="c", subcore_axis_name="s")

    @functools.partial(
        pl.kernel,
        out_type=jax.ShapeDtypeStruct((n_total, d), jnp.float32),
        mesh=mesh,
        scratch_types=[
            pltpu.VMEM((k_streams, IDX_W), jnp.int32),
            pltpu.VMEM((chunk, d), jnp.float32),
            pltpu.SemaphoreType.DMA,
        ],
    )
    def emb(idx_hbm, table_hbm, out_hbm, idx_v, rows_v, sem):
        wid = lax.axis_index("s") * nc + lax.axis_index("c")
        base = wid * per_w
        base_rows = wid * per_w_rows

        def body(g, carry):
            off = base + g * chunk
            pltpu.sync_copy(idx_hbm.at[pl.ds(base_rows + g * k_streams, k_streams)],
                            idx_v)
            for j in range(k_streams):
                for i in range(IDX_W // LANES):
                    sl = (j, pl.ds(i * LANES, LANES))
                    v = idx_v[sl]
                    idx_v[sl] = jnp.minimum(jnp.maximum(v, 0), vocab - 1)
            copies = [
                pltpu.async_copy(table_hbm.at[idx_v.at[j]],
                                 rows_v.at[pl.ds(j * IDX_W, IDX_W)], sem)
                for j in range(k_streams)
            ]
            for c in copies:
                c.wait()
            pltpu.sync_copy(rows_v, out_hbm.at[pl.ds(off, chunk)])
            return carry

        lax.fori_loop(0, n_chunks, body, 0)

    out = emb(idx2d, table)
    return out.reshape(batch, clicked, d)


# speedup vs baseline: 10.8415x; 1.4337x over previous
import functools

import jax
import jax.numpy as jnp
from jax import lax
from jax.experimental import pallas as pl
from jax.experimental.pallas import tpu as pltpu
from jax.experimental.pallas import tpu_sc as plsc

LANES = 16
IDX_W = 128
CHUNK = 256
NBUF = 2


def _sc_workers():
    try:
        info = plsc.get_sparse_core_info()
        return info.num_cores, info.num_subcores
    except Exception:
        return 2, 16


def kernel(ctr, table):
    batch, clicked = ctr.shape
    vocab, d = table.shape
    n_total = batch * clicked

    nc, ns = _sc_workers()
    nw = nc * ns
    per_w = n_total // nw
    assert per_w * nw == n_total and per_w % CHUNK == 0
    k_streams = CHUNK // IDX_W
    n_chunks = per_w // CHUNK
    assert n_chunks % NBUF == 0 and n_chunks >= 2 * NBUF
    n_outer = n_chunks // NBUF
    per_w_rows = per_w // IDX_W

    idx2d = ctr.reshape(n_total // IDX_W, IDX_W)

    mesh = plsc.VectorSubcoreMesh(core_axis_name="c", subcore_axis_name="s")

    @functools.partial(
        pl.kernel,
        out_type=jax.ShapeDtypeStruct((n_total, d), jnp.float32),
        mesh=mesh,
        scratch_types=[
            pltpu.VMEM((NBUF, k_streams, IDX_W), jnp.int32),
            pltpu.VMEM((NBUF, CHUNK, d), jnp.float32),
        ] + [pltpu.SemaphoreType.DMA] * (3 * NBUF),
    )
    def emb(idx_hbm, table_hbm, out_hbm, idx_v, rows_v, *sems):
        isem = sems[0:NBUF]
        gsem = sems[NBUF:2 * NBUF]
        osem = sems[2 * NBUF:3 * NBUF]
        wid = lax.axis_index("s") * nc + lax.axis_index("c")
        base = wid * per_w
        base_rows = wid * per_w_rows

        def idx_src(g):
            return idx_hbm.at[pl.ds(base_rows + g * k_streams, k_streams)]

        def out_dst(g):
            return out_hbm.at[pl.ds(base + g * CHUNK, CHUNK)]

        def start_idx(g, b):
            pltpu.async_copy(idx_src(g), idx_v.at[b], isem[b])

        def do_chunk(g, b, wait_out_g, prefetch_g):
            pltpu.make_async_copy(idx_src(g), idx_v.at[b], isem[b]).wait()
            for j in range(k_streams):
                for i in range(IDX_W // LANES):
                    sl = (b, j, pl.ds(i * LANES, LANES))
                    v = idx_v[sl]
                    idx_v[sl] = jnp.minimum(jnp.maximum(v, 0), vocab - 1)
            if wait_out_g is not None:
                pltpu.make_async_copy(rows_v.at[b], out_dst(wait_out_g),
                                      osem[b]).wait()
            copies = [
                pltpu.async_copy(table_hbm.at[idx_v.at[b, j]],
                                 rows_v.at[b, pl.ds(j * IDX_W, IDX_W)], gsem[b])
                for j in range(k_streams)
            ]
            for c in copies:
                c.wait()
            pltpu.async_copy(rows_v.at[b], out_dst(g), osem[b])
            if prefetch_g is not None:
                start_idx(prefetch_g, b)

        for b in range(NBUF):
            start_idx(b, b)
        for b in range(NBUF):
            do_chunk(b, b, None, NBUF + b)

        def body(outer, carry):
            g0 = outer * NBUF
            for b in range(NBUF):
                do_chunk(g0 + b, b, g0 + b - NBUF, g0 + b + NBUF)
            return carry

        lax.fori_loop(1, n_outer - 1, body, 0)

        g0 = (n_outer - 1) * NBUF
        for b in range(NBUF):
            do_chunk(g0 + b, b, g0 + b - NBUF, None)
        for b in range(NBUF):
            pltpu.make_async_copy(rows_v.at[b], out_dst(g0 + b), osem[b]).wait()

    out = emb(idx2d, table)
    return out.reshape(batch, clicked, d)
